# baseline (speedup 1.0000x reference)
import jax
import jax.numpy as jnp
from jax import lax
from jax.experimental import pallas as pl
from jax.experimental.pallas import tpu as pltpu
from jax.experimental.pallas import tpu_sc as plsc

D_EMBED = 64
BATCH = 16384
N_NEG = 5

_NC = 2
_NS = 16
_NW = _NC * _NS
_L = 128
_NBUF = 4

_PC_CH = BATCH // _NW // _L
_CX_CH = BATCH * (1 + N_NEG) // _NW // _L


def _body(wc_hbm, wx_hbm, pc_idx, cx_idx, out_pc, out_cx,
          idx_v, rows_v, gsem, wsem):
  wid = lax.axis_index("s") * _NC + lax.axis_index("c")

  pltpu.sync_copy(pc_idx.at[pl.ds(wid * _PC_CH, _PC_CH)],
                  idx_v.at[pl.ds(0, _PC_CH)])
  pltpu.sync_copy(cx_idx.at[pl.ds(wid * _CX_CH, _CX_CH)],
                  idx_v.at[pl.ds(_PC_CH, _CX_CH)])

  def phase(tbl, out, idx_off, n_ch):
    cbase = wid * n_ch

    def group(g):
      cps = []
      for b in range(_NBUF):
        cp = pltpu.make_async_copy(
            tbl.at[idx_v.at[idx_off + g + b]], rows_v.at[b], gsem.at[b])
        cp.start()
        cps.append(cp)
      wps = []
      for b in range(_NBUF):
        cps[b].wait()
        wp = pltpu.make_async_copy(
            rows_v.at[b], out.at[pl.ds((cbase + g + b) * _L, _L)],
            wsem.at[b])
        wp.start()
        wps.append(wp)
      for b in range(_NBUF):
        wps[b].wait()

    if n_ch <= _NBUF:
      group(0)
    else:
      lax.fori_loop(0, n_ch // _NBUF, lambda i, c: (group(i * _NBUF), c)[1],
                    0)

  phase(wc_hbm, out_pc, 0, _PC_CH)
  phase(wx_hbm, out_cx, _PC_CH, _CX_CH)


@jax.jit
def _gather(W_center, W_context, pc_idx, cx_idx):
  run = pl.kernel(
      _body,
      out_type=(
          jax.ShapeDtypeStruct((BATCH, D_EMBED), jnp.float32),
          jax.ShapeDtypeStruct((BATCH * (1 + N_NEG), D_EMBED), jnp.float32),
      ),
      mesh=plsc.VectorSubcoreMesh(core_axis_name="c", subcore_axis_name="s"),
      scratch_types=[
          pltpu.VMEM((_PC_CH + _CX_CH, _L), jnp.int32),
          pltpu.VMEM((_NBUF, _L, D_EMBED), jnp.float32),
          pltpu.SemaphoreType.DMA((_NBUF,)),
          pltpu.SemaphoreType.DMA((_NBUF,)),
      ],
      compiler_params=pltpu.CompilerParams(use_tc_tiling_on_sc=False),
  )
  out_pc, out_cx = run(W_center, W_context, pc_idx, cx_idx)
  px = out_cx[:BATCH]
  nx = out_cx[BATCH:].reshape(BATCH, N_NEG, D_EMBED)
  return out_pc, px, nx


def kernel(W_center, W_context, pos_center, pos_context, neg_context):
  pc_idx = pos_center.astype(jnp.int32).reshape(-1, _L)
  cx_idx = jnp.concatenate(
      [pos_context.astype(jnp.int32),
       neg_context.astype(jnp.int32).reshape(-1)]).reshape(-1, _L)
  return _gather(W_center, W_context, pc_idx, cx_idx)
